# Initial kernel scaffold; baseline (speedup 1.0000x reference)
import jax
import jax.numpy as jnp
from jax.experimental import pallas as pl


def kernel(pos, flat_netpin, netpin_start, net_mask):
    raise NotImplementedError("write your pallas kernel here")


# speedup vs baseline: 1.3222x; 1.3222x over previous
import functools

import jax
import jax.numpy as jnp
from jax import lax
from jax.experimental import pallas as pl
from jax.experimental.pallas import tpu as pltpu
from jax.experimental.pallas import tpu_sc as plsc

NUM_PINS = 32768
NUM_NETS = 2048
LAM = 1.0
MU_W = 1.0
SIG = 1.0

NW = 32
NPW = NUM_NETS // NW
LANES = 16

TM = 256
TN = 256
NB = NUM_NETS // TM


def _gather_body(pos_hbm, fnp_hbm, s0_hbm, s1_hbm, mask_hbm, out_hbm,
                 s_v, e_v, m_v, pa_v, pb_v, ia_v, ib_v, pins_a, pins_b,
                 x1_v, y1_v, x2_v, y2_v, ra_v, rb_v, rc_v, rv_v, sem):
    cid = lax.axis_index("c")
    sid = lax.axis_index("s")
    wid = cid * 16 + sid
    base = wid * NPW

    pltpu.sync_copy(s0_hbm.at[pl.ds(base, NPW)], s_v)
    pltpu.sync_copy(s1_hbm.at[pl.ds(base, NPW)], e_v)
    pltpu.sync_copy(mask_hbm.at[pl.ds(base, NPW)], m_v)

    for j in range(NPW // LANES):
        sl = pl.ds(j * LANES, LANES)
        st = s_v[sl]
        deg = e_v[sl] - st
        ok = (deg >= 2) & (m_v[sl] > 0)
        rv_v[sl] = jnp.where(ok, 1.0, 0.0).astype(jnp.float32)
        pa_v[sl] = jnp.clip(st, 0, NUM_PINS - 1)
        pb_v[sl] = jnp.clip(st + 1, 0, NUM_PINS - 1)

    pltpu.async_copy(fnp_hbm.at[pa_v], pins_a, sem).wait()
    pltpu.async_copy(fnp_hbm.at[pb_v], pins_b, sem).wait()

    for j in range(NPW // LANES):
        sl = pl.ds(j * LANES, LANES)
        ia_v[sl] = pins_a[sl] + NUM_PINS
        ib_v[sl] = pins_b[sl] + NUM_PINS

    pltpu.async_copy(pos_hbm.at[pins_a], x1_v, sem).wait()
    pltpu.async_copy(pos_hbm.at[ia_v], y1_v, sem).wait()
    pltpu.async_copy(pos_hbm.at[pins_b], x2_v, sem).wait()
    pltpu.async_copy(pos_hbm.at[ib_v], y2_v, sem).wait()

    for j in range(NPW // LANES):
        sl = pl.ds(j * LANES, LANES)
        x1 = x1_v[sl]
        y1 = y1_v[sl]
        a = x2_v[sl] - x1
        b = y2_v[sl] - y1
        ra_v[sl] = a
        rb_v[sl] = b
        rc_v[sl] = a * y1 - b * x1

    rows = (x1_v, y1_v, x2_v, y2_v, ra_v, rb_v, rc_v, rv_v)
    for r, buf in enumerate(rows):
        pltpu.sync_copy(buf, out_hbm.at[pl.ds(r * NUM_NETS + base, NPW)])


def _make_gather_call():
    mesh = plsc.VectorSubcoreMesh(core_axis_name="c", subcore_axis_name="s")
    return functools.partial(
        pl.kernel,
        mesh=mesh,
        out_type=jax.ShapeDtypeStruct((8 * NUM_NETS,), jnp.float32),
        scratch_types=[
        pltpu.VMEM((NPW,), jnp.int32),
        pltpu.VMEM((NPW,), jnp.int32),
        pltpu.VMEM((NPW,), jnp.int32),
        pltpu.VMEM((NPW,), jnp.int32),
        pltpu.VMEM((NPW,), jnp.int32),
        pltpu.VMEM((NPW,), jnp.int32),
        pltpu.VMEM((NPW,), jnp.int32),
        pltpu.VMEM((NPW,), jnp.int32),
        pltpu.VMEM((NPW,), jnp.int32),
        pltpu.VMEM((NPW,), jnp.float32),
        pltpu.VMEM((NPW,), jnp.float32),
        pltpu.VMEM((NPW,), jnp.float32),
        pltpu.VMEM((NPW,), jnp.float32),
        pltpu.VMEM((NPW,), jnp.float32),
        pltpu.VMEM((NPW,), jnp.float32),
        pltpu.VMEM((NPW,), jnp.float32),
            pltpu.VMEM((NPW,), jnp.float32),
            pltpu.SemaphoreType.DMA,
        ],
    )(_gather_body)


def _pair_body(p_ref, pt_ref, out_ref):
    ib = pl.program_id(0)
    jb = pl.program_id(1)

    @pl.when(jnp.logical_and(ib == 0, jb == 0))
    def _():
        out_ref[0, 0] = 0.0

    @pl.when(jb >= ib)
    def _():
        x1j = p_ref[pl.ds(0, 1), :]
        y1j = p_ref[pl.ds(1, 1), :]
        x2j = p_ref[pl.ds(2, 1), :]
        y2j = p_ref[pl.ds(3, 1), :]
        aj = p_ref[pl.ds(4, 1), :]
        bj = p_ref[pl.ds(5, 1), :]
        cj = p_ref[pl.ds(6, 1), :]
        vj = p_ref[pl.ds(7, 1), :]
        x1i = pt_ref[:, pl.ds(0, 1)]
        y1i = pt_ref[:, pl.ds(1, 1)]
        x2i = pt_ref[:, pl.ds(2, 1)]
        y2i = pt_ref[:, pl.ds(3, 1)]
        ai = pt_ref[:, pl.ds(4, 1)]
        bi = pt_ref[:, pl.ds(5, 1)]
        ci = pt_ref[:, pl.ds(6, 1)]
        vi = pt_ref[:, pl.ds(7, 1)]

        d1 = ai * y1j - bi * x1j - ci
        d2 = ai * y2j - bi * x2j - ci
        d1t = aj * y1i - bj * x1i - cj
        d2t = aj * y2i - bj * x2i - cj

        s1 = 1.0 / (1.0 + jnp.exp(-LAM * d1))
        s2 = 1.0 / (1.0 + jnp.exp(-LAM * d2))
        s3 = 1.0 / (1.0 + jnp.exp(-LAM * d1t))
        s4 = 1.0 / (1.0 + jnp.exp(-LAM * d2t))
        opp = s1 + s2 - 2.0 * s1 * s2
        oppt = s3 + s4 - 2.0 * s3 * s4

        inv2s2 = 0.5 / (SIG * SIG)
        bell = jnp.exp(-(d1 * d1 + d2 * d2) * inv2s2)
        bellt = jnp.exp(-(d1t * d1t + d2t * d2t) * inv2s2)

        score = opp * oppt + MU_W * bell * bellt

        grow = ib * TM + lax.broadcasted_iota(jnp.int32, (TM, TN), 0)
        gcol = jb * TN + lax.broadcasted_iota(jnp.int32, (TM, TN), 1)
        w = jnp.where(gcol > grow, vi * vj, 0.0)
        out_ref[0, 0] += jnp.sum(score * w)


def _pair_call(pm, pt):
    return pl.pallas_call(
        _pair_body,
        grid=(NB, NB),
        in_specs=[
            pl.BlockSpec((8, TN), lambda i, j: (0, j)),
            pl.BlockSpec((TM, 8), lambda i, j: (i, 0)),
        ],
        out_specs=pl.BlockSpec(memory_space=pltpu.SMEM),
        out_shape=jax.ShapeDtypeStruct((1, 1), jnp.float32),
    )(pm, pt)


def kernel(pos, flat_netpin, netpin_start, net_mask):
    s0 = netpin_start[:-1]
    s1 = netpin_start[1:]
    mask_i = net_mask.astype(jnp.int32)
    feats = _make_gather_call()(pos, flat_netpin, s0, s1, mask_i)
    pm = feats.reshape(8, NUM_NETS)
    pt = pm.T
    return _pair_call(pm, pt)[0, 0]


# speedup vs baseline: 1.3598x; 1.0285x over previous
import functools

import jax
import jax.numpy as jnp
from jax import lax
from jax.experimental import pallas as pl
from jax.experimental.pallas import tpu as pltpu
from jax.experimental.pallas import tpu_sc as plsc

NUM_PINS = 32768
NUM_NETS = 2048
LAM = 1.0
MU_W = 1.0
SIG = 1.0

NW = 32
NPW = NUM_NETS // NW
LANES = 16

TM = 256
TN = 256
NB = NUM_NETS // TM


def _gather_body(pos_hbm, fnp_hbm, s0_hbm, s1_hbm, mask_hbm, out_hbm,
                 s_v, e_v, m_v, pa_v, pb_v, ia_v, ib_v, pins_a, pins_b,
                 x1_v, y1_v, x2_v, y2_v, ra_v, rb_v, rc_v, rv_v, sem):
    cid = lax.axis_index("c")
    sid = lax.axis_index("s")
    wid = cid * 16 + sid
    base = wid * NPW

    pltpu.sync_copy(s0_hbm.at[pl.ds(base, NPW)], s_v)
    pltpu.sync_copy(s1_hbm.at[pl.ds(base, NPW)], e_v)
    pltpu.sync_copy(mask_hbm.at[pl.ds(base, NPW)], m_v)

    for j in range(NPW // LANES):
        sl = pl.ds(j * LANES, LANES)
        st = s_v[sl]
        deg = e_v[sl] - st
        ok = (deg >= 2) & (m_v[sl] > 0)
        rv_v[sl] = jnp.where(ok, 1.0, 0.0).astype(jnp.float32)
        pa_v[sl] = jnp.clip(st, 0, NUM_PINS - 1)
        pb_v[sl] = jnp.clip(st + 1, 0, NUM_PINS - 1)

    pltpu.async_copy(fnp_hbm.at[pa_v], pins_a, sem).wait()
    pltpu.async_copy(fnp_hbm.at[pb_v], pins_b, sem).wait()

    for j in range(NPW // LANES):
        sl = pl.ds(j * LANES, LANES)
        ia_v[sl] = pins_a[sl] + NUM_PINS
        ib_v[sl] = pins_b[sl] + NUM_PINS

    pltpu.async_copy(pos_hbm.at[pins_a], x1_v, sem).wait()
    pltpu.async_copy(pos_hbm.at[ia_v], y1_v, sem).wait()
    pltpu.async_copy(pos_hbm.at[pins_b], x2_v, sem).wait()
    pltpu.async_copy(pos_hbm.at[ib_v], y2_v, sem).wait()

    for j in range(NPW // LANES):
        sl = pl.ds(j * LANES, LANES)
        x1 = x1_v[sl]
        y1 = y1_v[sl]
        a = x2_v[sl] - x1
        b = y2_v[sl] - y1
        ok = rv_v[sl] > 0.5
        ra_v[sl] = jnp.where(ok, a, 0.0)
        rb_v[sl] = jnp.where(ok, b, 0.0)
        rc_v[sl] = jnp.where(ok, a * y1 - b * x1, -1e6)

    rows = (x1_v, y1_v, x2_v, y2_v, ra_v, rb_v, rc_v, rv_v)
    for r, buf in enumerate(rows):
        pltpu.sync_copy(buf, out_hbm.at[pl.ds(r * NUM_NETS + base, NPW)])


def _make_gather_call():
    mesh = plsc.VectorSubcoreMesh(core_axis_name="c", subcore_axis_name="s")
    return functools.partial(
        pl.kernel,
        mesh=mesh,
        out_type=jax.ShapeDtypeStruct((8 * NUM_NETS,), jnp.float32),
        scratch_types=[
        pltpu.VMEM((NPW,), jnp.int32),
        pltpu.VMEM((NPW,), jnp.int32),
        pltpu.VMEM((NPW,), jnp.int32),
        pltpu.VMEM((NPW,), jnp.int32),
        pltpu.VMEM((NPW,), jnp.int32),
        pltpu.VMEM((NPW,), jnp.int32),
        pltpu.VMEM((NPW,), jnp.int32),
        pltpu.VMEM((NPW,), jnp.int32),
        pltpu.VMEM((NPW,), jnp.int32),
        pltpu.VMEM((NPW,), jnp.float32),
        pltpu.VMEM((NPW,), jnp.float32),
        pltpu.VMEM((NPW,), jnp.float32),
        pltpu.VMEM((NPW,), jnp.float32),
        pltpu.VMEM((NPW,), jnp.float32),
        pltpu.VMEM((NPW,), jnp.float32),
        pltpu.VMEM((NPW,), jnp.float32),
            pltpu.VMEM((NPW,), jnp.float32),
            pltpu.SemaphoreType.DMA,
        ],
    )(_gather_body)


def _pair_body(p_ref, pt_ref, out_ref):
    ib = pl.program_id(0)
    jb = pl.program_id(1)

    @pl.when(jnp.logical_and(ib == 0, jb == 0))
    def _():
        out_ref[0, 0] = 0.0

    @pl.when(jb >= ib)
    def _():
        x1j = p_ref[pl.ds(0, 1), :]
        y1j = p_ref[pl.ds(1, 1), :]
        x2j = p_ref[pl.ds(2, 1), :]
        y2j = p_ref[pl.ds(3, 1), :]
        aj = p_ref[pl.ds(4, 1), :]
        bj = p_ref[pl.ds(5, 1), :]
        cj = p_ref[pl.ds(6, 1), :]
        x1i = pt_ref[:, pl.ds(0, 1)]
        y1i = pt_ref[:, pl.ds(1, 1)]
        x2i = pt_ref[:, pl.ds(2, 1)]
        y2i = pt_ref[:, pl.ds(3, 1)]
        ai = pt_ref[:, pl.ds(4, 1)]
        bi = pt_ref[:, pl.ds(5, 1)]
        ci = pt_ref[:, pl.ds(6, 1)]

        d1 = ai * y1j - bi * x1j - ci
        d2 = ai * y2j - bi * x2j - ci
        d1t = aj * y1i - bj * x1i - cj
        d2t = aj * y2i - bj * x2i - cj

        u = jnp.exp(jnp.minimum(-LAM * d1, 20.0))
        v = jnp.exp(jnp.minimum(-LAM * d2, 20.0))
        ut = jnp.exp(jnp.minimum(-LAM * d1t, 20.0))
        vt = jnp.exp(jnp.minimum(-LAM * d2t, 20.0))
        num = (u + v) * (ut + vt)
        den = ((1.0 + u) * (1.0 + v)) * ((1.0 + ut) * (1.0 + vt))
        cross = num / den

        inv2s2 = 0.5 / (SIG * SIG)
        bell2 = jnp.exp(-((d1 * d1 + d2 * d2) +
                          (d1t * d1t + d2t * d2t)) * inv2s2)

        tile_sum = jnp.sum(cross + MU_W * bell2)
        sv = jnp.sum(pt_ref[:, pl.ds(7, 1)])
        contrib = jnp.where(ib == jb, 0.5 * (tile_sum - 1.25 * sv),
                            tile_sum)
        out_ref[0, 0] += contrib


def _pair_call(pm, pt):
    return pl.pallas_call(
        _pair_body,
        grid=(NB, NB),
        in_specs=[
            pl.BlockSpec((8, TN), lambda i, j: (0, j)),
            pl.BlockSpec((TM, 8), lambda i, j: (i, 0)),
        ],
        out_specs=pl.BlockSpec(memory_space=pltpu.SMEM),
        out_shape=jax.ShapeDtypeStruct((1, 1), jnp.float32),
    )(pm, pt)


def kernel(pos, flat_netpin, netpin_start, net_mask):
    s0 = netpin_start[:-1]
    s1 = netpin_start[1:]
    mask_i = net_mask.astype(jnp.int32)
    feats = _make_gather_call()(pos, flat_netpin, s0, s1, mask_i)
    pm = feats.reshape(8, NUM_NETS)
    pt = pm.T
    return _pair_call(pm, pt)[0, 0]
